# Initial kernel scaffold; baseline (speedup 1.0000x reference)
import jax
import jax.numpy as jnp
from jax.experimental import pallas as pl


def kernel(x, edge_index, edge_attr, params):
    raise NotImplementedError("write your pallas kernel here")


# speedup vs baseline: 1.0750x; 1.0750x over previous
import functools

import jax
import jax.numpy as jnp
from jax.experimental import pallas as pl
from jax.experimental.pallas import tpu as pltpu

N_NODES = 50000
N_EDGES = 800000
D = 64
R = 8

_ROW_TILE = 1000


def _ln_gelu_body(h_ref, g_ref, b_ref, o_ref):
    h = h_ref[...]
    m = jnp.mean(h, axis=-1, keepdims=True)
    v = jnp.mean((h - m) * (h - m), axis=-1, keepdims=True)
    xn = (h - m) * jax.lax.rsqrt(v + 1e-5) * g_ref[...] + b_ref[...]
    o_ref[...] = xn * 0.5 * (1.0 + jax.lax.erf(xn / jnp.sqrt(2.0).astype(xn.dtype)))


def _ln_gelu(h, g, b):
    grid = (N_NODES // _ROW_TILE,)
    return pl.pallas_call(
        _ln_gelu_body,
        grid=grid,
        in_specs=[
            pl.BlockSpec((_ROW_TILE, D), lambda i: (i, 0)),
            pl.BlockSpec((1, D), lambda i: (0, 0)),
            pl.BlockSpec((1, D), lambda i: (0, 0)),
        ],
        out_specs=pl.BlockSpec((_ROW_TILE, D), lambda i: (i, 0)),
        out_shape=jax.ShapeDtypeStruct((N_NODES, D), jnp.float32),
    )(h, g.reshape(1, D), b.reshape(1, D))


def _rgcn_conv(y, src, dst, etype, invE, W, root, b):
    xW = jnp.einsum('ni,rio->rno', y, W).reshape(R * N_NODES, D)
    msg = xW[etype * N_NODES + src]
    agg = jnp.zeros((N_NODES, D), jnp.float32).at[dst].add(msg * invE[:, None])
    return agg + y @ root + b


def kernel(x, edge_index, edge_attr, params):
    src = edge_index[0]
    dst = edge_index[1]
    et = edge_attr
    comb = dst * R + et
    cnt = jnp.zeros(N_NODES * R, jnp.float32).at[comb].add(1.0)
    inv = 1.0 / jnp.maximum(cnt, 1.0)
    invE = inv[comb]

    emb = params['emb']
    h = jnp.where((x[:, None] == 1), emb[1][None, :], emb[0][None, :])

    for p in params['blocks']:
        y = _ln_gelu(h, p['ln1_g'], p['ln1_b'])
        out = _rgcn_conv(y, src, dst, et, invE, p['W1'], p['root1'], p['b1'])
        y2 = _ln_gelu(out, p['ln2_g'], p['ln2_b'])
        out2 = _rgcn_conv(y2, src, dst, et, invE, p['W2'], p['root2'], p['b2'])
        h = out2 + h

    h = h @ params['cr_W'] + params['cr_b']
    pooled = jnp.mean(h, axis=0)
    z = jax.nn.gelu(pooled @ params['p1_W'] + params['p1_b'], approximate=False)
    return z @ params['p2_W'] + params['p2_b']


# speedup vs baseline: 2.0700x; 1.9257x over previous
import functools

import jax
import jax.numpy as jnp
from jax import lax
from jax.experimental import pallas as pl
from jax.experimental.pallas import tpu as pltpu
from jax.experimental.pallas import tpu_sc as plsc

N_NODES = 50000
N_EDGES = 800000
D = 64
R = 8
DH = D // 2

NC = 2
NS = 16
L = 16

EB = 128
EPT = 392 * EB
EPAD = NS * EPT
NB = EPT // EB

AGG_ROWS = 50176
PER_S = AGG_ROWS // NS
CNT_ROWS = 401408
CNT_PER_S = CNT_ROWS // NS
EPW = EPAD // (NC * NS)
NBW = EPW // EB

_ROW_TILE = 1000

_mesh = plsc.VectorSubcoreMesh(core_axis_name="c", subcore_axis_name="s")

_BCAST_DNUMS = lax.GatherDimensionNumbers(
    offset_dims=(), collapsed_slice_dims=(0,), start_index_map=(0,))


def _lane_bcast(v, j):
    idx = jnp.full((L, 1), j, jnp.int32)
    return lax.gather(v, idx, _BCAST_DNUMS, (1,),
                      mode=lax.GatherScatterMode.PROMISE_IN_BOUNDS)


def _sc_count(comb, val, zblk):
    @functools.partial(
        pl.kernel,
        out_type=jax.ShapeDtypeStruct((CNT_ROWS,), jnp.float32),
        mesh=_mesh,
        compiler_params=pltpu.CompilerParams(use_tc_tiling_on_sc=False),
        scratch_types=[
            pltpu.VMEM_SHARED((CNT_ROWS,), jnp.float32),
            pltpu.VMEM((EB,), jnp.int32),
            pltpu.VMEM((EB,), jnp.float32),
        ],
    )
    def k(comb_h, val_h, z_h, out_h, cnt_sh, comb_v, val_v):
        c = lax.axis_index("c")
        s = lax.axis_index("s")

        @pl.when(c == 0)
        def _():
            pltpu.sync_copy(z_h, cnt_sh.at[pl.ds(s * CNT_PER_S, CNT_PER_S)])
            plsc.subcore_barrier()

            def body(b, _):
                e0 = s * EPT + b * EB
                pltpu.sync_copy(comb_h.at[pl.ds(e0, EB)], comb_v)
                pltpu.sync_copy(val_h.at[pl.ds(e0, EB)], val_v)
                pltpu.sync_copy(val_v, cnt_sh.at[comb_v], add=True)
                return ()

            lax.fori_loop(0, NB, body, ())
            plsc.subcore_barrier()
            pltpu.sync_copy(
                cnt_sh.at[pl.ds(s * CNT_PER_S, CNT_PER_S)],
                out_h.at[pl.ds(s * CNT_PER_S, CNT_PER_S)],
            )

    return k(comb, val, zblk)


def _sc_inv_gather(comb, invp):
    @functools.partial(
        pl.kernel,
        out_type=jax.ShapeDtypeStruct((EPAD,), jnp.float32),
        mesh=_mesh,
        compiler_params=pltpu.CompilerParams(use_tc_tiling_on_sc=False),
        scratch_types=[
            pltpu.VMEM((EB,), jnp.int32),
            pltpu.VMEM((EB,), jnp.float32),
        ],
    )
    def k(comb_h, invp_h, out_h, comb_v, inv_v):
        c = lax.axis_index("c")
        s = lax.axis_index("s")
        base = (s * NC + c) * EPW

        def body(b, _):
            e0 = base + b * EB
            pltpu.sync_copy(comb_h.at[pl.ds(e0, EB)], comb_v)
            pltpu.sync_copy(invp_h.at[comb_v], inv_v)
            pltpu.sync_copy(inv_v, out_h.at[pl.ds(e0, EB)])
            return ()

        lax.fori_loop(0, NBW, body, ())

    return k(comb, invp)


def _sc_conv(tbl, g2, dstv, invE, zblk):
    @functools.partial(
        pl.kernel,
        out_type=jax.ShapeDtypeStruct((NC, AGG_ROWS, DH), jnp.float32),
        mesh=_mesh,
        compiler_params=pltpu.CompilerParams(use_tc_tiling_on_sc=False),
        scratch_types=[
            pltpu.VMEM_SHARED((AGG_ROWS, DH), jnp.float32),
            pltpu.VMEM((EB,), jnp.int32),
            pltpu.VMEM((EB,), jnp.int32),
            pltpu.VMEM((EB,), jnp.float32),
            pltpu.VMEM((EB,), jnp.int32),
            pltpu.VMEM((EB, DH), jnp.float32),
        ],
    )
    def k(tbl_h, g2_h, dst_h, inv_h, z_h, out_h,
          agg_sh, g2_v, dst_v, inv_v, idx_v, rows_v):
        c = lax.axis_index("c")
        s = lax.axis_index("s")

        pltpu.sync_copy(z_h, agg_sh.at[pl.ds(s * PER_S, PER_S)])
        plsc.subcore_barrier()

        def body(b, _):
            e0 = s * EPT + b * EB
            pltpu.sync_copy(g2_h.at[pl.ds(e0, EB)], g2_v)
            pltpu.sync_copy(dst_h.at[pl.ds(e0, EB)], dst_v)
            pltpu.sync_copy(inv_h.at[pl.ds(e0, EB)], inv_v)
            for kk in range(EB // L):
                idx_v[pl.ds(kk * L, L)] = g2_v[pl.ds(kk * L, L)] + c
            pltpu.sync_copy(tbl_h.at[idx_v], rows_v)
            for kk in range(EB // L):
                iv = inv_v[pl.ds(kk * L, L)]
                for j in range(L):
                    e = kk * L + j
                    bc = _lane_bcast(iv, j)
                    rows_v[e, pl.ds(0, L)] = rows_v[e, pl.ds(0, L)] * bc
                    rows_v[e, pl.ds(L, L)] = rows_v[e, pl.ds(L, L)] * bc
            pltpu.sync_copy(rows_v, agg_sh.at[dst_v], add=True)
            return ()

        lax.fori_loop(0, NB, body, ())
        plsc.subcore_barrier()
        pltpu.sync_copy(
            agg_sh.at[pl.ds(s * PER_S, PER_S)],
            out_h.at[c, pl.ds(s * PER_S, PER_S)],
        )

    return k(tbl, g2, dstv, invE, zblk)


def _ln_gelu_body(h_ref, g_ref, b_ref, o_ref):
    h = h_ref[...]
    m = jnp.mean(h, axis=-1, keepdims=True)
    v = jnp.mean((h - m) * (h - m), axis=-1, keepdims=True)
    xn = (h - m) * jax.lax.rsqrt(v + 1e-5) * g_ref[...] + b_ref[...]
    o_ref[...] = xn * 0.5 * (1.0 + jax.lax.erf(xn / jnp.sqrt(2.0).astype(xn.dtype)))


def _ln_gelu(h, g, b):
    grid = (N_NODES // _ROW_TILE,)
    return pl.pallas_call(
        _ln_gelu_body,
        grid=grid,
        in_specs=[
            pl.BlockSpec((_ROW_TILE, D), lambda i: (i, 0)),
            pl.BlockSpec((1, D), lambda i: (0, 0)),
            pl.BlockSpec((1, D), lambda i: (0, 0)),
        ],
        out_specs=pl.BlockSpec((_ROW_TILE, D), lambda i: (i, 0)),
        out_shape=jax.ShapeDtypeStruct((N_NODES, D), jnp.float32),
    )(h, g.reshape(1, D), b.reshape(1, D))


def _rgcn_conv(y, g2, dstv, invE, zagg, W, root, b):
    xW = jnp.einsum('ni,rio->rno', y, W)
    tbl = xW.reshape(R * N_NODES * NC, DH)
    agg3 = _sc_conv(tbl, g2, dstv, invE, zagg)
    agg = jnp.concatenate([agg3[0][:N_NODES], agg3[1][:N_NODES]], axis=-1)
    return agg + y @ root + b


def kernel(x, edge_index, edge_attr, params):
    src = edge_index[0].astype(jnp.int32)
    dst = edge_index[1].astype(jnp.int32)
    et = edge_attr.astype(jnp.int32)

    pad = EPAD - N_EDGES
    comb = jnp.concatenate([dst * R + et, jnp.full((pad,), N_NODES * R, jnp.int32)])
    g2 = jnp.concatenate([(et * N_NODES + src) * NC, jnp.zeros((pad,), jnp.int32)])
    dstp = jnp.concatenate([dst, jnp.full((pad,), N_NODES, jnp.int32)])
    val = jnp.concatenate([jnp.ones((N_EDGES,), jnp.float32),
                           jnp.zeros((pad,), jnp.float32)])

    zcnt = jnp.zeros((CNT_PER_S,), jnp.float32)
    zagg = jnp.zeros((PER_S, DH), jnp.float32)

    cnt = _sc_count(comb, val, zcnt)
    invp = 1.0 / jnp.maximum(cnt, 1.0)
    invE = _sc_inv_gather(comb, invp)

    emb = params['emb']
    h = jnp.where((x[:, None] == 1), emb[1][None, :], emb[0][None, :])

    for p in params['blocks']:
        y = _ln_gelu(h, p['ln1_g'], p['ln1_b'])
        out = _rgcn_conv(y, g2, dstp, invE, zagg, p['W1'], p['root1'], p['b1'])
        y2 = _ln_gelu(out, p['ln2_g'], p['ln2_b'])
        out2 = _rgcn_conv(y2, g2, dstp, invE, zagg, p['W2'], p['root2'], p['b2'])
        h = out2 + h

    h = h @ params['cr_W'] + params['cr_b']
    pooled = jnp.mean(h, axis=0)
    z = jax.nn.gelu(pooled @ params['p1_W'] + params['p1_b'], approximate=False)
    return z @ params['p2_W'] + params['p2_b']


# speedup vs baseline: 4.0798x; 1.9709x over previous
import functools

import jax
import jax.numpy as jnp
from jax import lax
from jax.experimental import pallas as pl
from jax.experimental.pallas import tpu as pltpu
from jax.experimental.pallas import tpu_sc as plsc

N_NODES = 50000
N_EDGES = 800000
D = 64
R = 8
DH = D // 2

NC = 2
NS = 16
L = 16

EB = 128
EPT = 392 * EB
EPAD = NS * EPT
NB = EPT // EB

AGG_ROWS = 50176
PER_S = AGG_ROWS // NS
CNT_ROWS = 401408
CNT_PER_S = CNT_ROWS // NS
EPW = EPAD // (NC * NS)
NBW = EPW // EB

_ROW_TILE = 1000

_mesh = plsc.VectorSubcoreMesh(core_axis_name="c", subcore_axis_name="s")

_BCAST_DNUMS = lax.GatherDimensionNumbers(
    offset_dims=(), collapsed_slice_dims=(0,), start_index_map=(0,))


def _lane_bcast(v, j):
    idx = jnp.full((L, 1), j, jnp.int32)
    return lax.gather(v, idx, _BCAST_DNUMS, (1,),
                      mode=lax.GatherScatterMode.PROMISE_IN_BOUNDS)


def _sc_count(comb, val, zblk):
    @functools.partial(
        pl.kernel,
        out_type=jax.ShapeDtypeStruct((CNT_ROWS,), jnp.float32),
        mesh=_mesh,
        compiler_params=pltpu.CompilerParams(use_tc_tiling_on_sc=False),
        scratch_types=[
            pltpu.VMEM_SHARED((CNT_ROWS,), jnp.float32),
            pltpu.VMEM((EB,), jnp.int32),
            pltpu.VMEM((EB,), jnp.float32),
        ],
    )
    def k(comb_h, val_h, z_h, out_h, cnt_sh, comb_v, val_v):
        c = lax.axis_index("c")
        s = lax.axis_index("s")

        @pl.when(c == 0)
        def _():
            pltpu.sync_copy(z_h, cnt_sh.at[pl.ds(s * CNT_PER_S, CNT_PER_S)])
            plsc.subcore_barrier()

            def body(b, _):
                e0 = s * EPT + b * EB
                pltpu.sync_copy(comb_h.at[pl.ds(e0, EB)], comb_v)
                pltpu.sync_copy(val_h.at[pl.ds(e0, EB)], val_v)
                pltpu.sync_copy(val_v, cnt_sh.at[comb_v], add=True)
                return ()

            lax.fori_loop(0, NB, body, ())
            plsc.subcore_barrier()
            pltpu.sync_copy(
                cnt_sh.at[pl.ds(s * CNT_PER_S, CNT_PER_S)],
                out_h.at[pl.ds(s * CNT_PER_S, CNT_PER_S)],
            )

    return k(comb, val, zblk)


def _sc_inv_gather(comb, invp):
    @functools.partial(
        pl.kernel,
        out_type=jax.ShapeDtypeStruct((EPAD,), jnp.float32),
        mesh=_mesh,
        compiler_params=pltpu.CompilerParams(use_tc_tiling_on_sc=False),
        scratch_types=[
            pltpu.VMEM((EB,), jnp.int32),
            pltpu.VMEM((EB,), jnp.float32),
        ],
    )
    def k(comb_h, invp_h, out_h, comb_v, inv_v):
        c = lax.axis_index("c")
        s = lax.axis_index("s")
        base = (s * NC + c) * EPW

        def body(b, _):
            e0 = base + b * EB
            pltpu.sync_copy(comb_h.at[pl.ds(e0, EB)], comb_v)
            pltpu.sync_copy(invp_h.at[comb_v], inv_v)
            pltpu.sync_copy(inv_v, out_h.at[pl.ds(e0, EB)])
            return ()

        lax.fori_loop(0, NBW, body, ())

    return k(comb, invp)


def _sc_conv(tbl, epk, invE, zblk):
    @functools.partial(
        pl.kernel,
        out_type=jax.ShapeDtypeStruct((NC, AGG_ROWS, DH), jnp.float32),
        mesh=_mesh,
        compiler_params=pltpu.CompilerParams(use_tc_tiling_on_sc=False),
        scratch_types=[
            pltpu.VMEM_SHARED((AGG_ROWS, DH), jnp.float32),
            pltpu.VMEM((4, 2 * EB), jnp.int32),
            pltpu.VMEM((4, EB), jnp.float32),
            pltpu.VMEM((4, EB), jnp.int32),
            pltpu.VMEM((4, EB), jnp.int32),
            pltpu.VMEM((2, EB, DH), jnp.float32),
            pltpu.VMEM((2, EB, DH), jnp.float32),
            pltpu.SemaphoreType.DMA((4,)),
            pltpu.SemaphoreType.DMA((4,)),
            pltpu.SemaphoreType.DMA((4,)),
        ],
    )
    def k(tbl_h, epk_h, inv_h, z_h, out_h,
          agg_sh, in_v, inf_v, idx_v, sidx_v, rows_v, srow_v,
          insem, gsem, ssem):
        c = lax.axis_index("c")
        s = lax.axis_index("s")

        pltpu.sync_copy(z_h, agg_sh.at[pl.ds(s * PER_S, PER_S)])
        plsc.subcore_barrier()

        def issue_in(b, j):
            e0 = (s * NB + b) * (2 * EB)
            pltpu.async_copy(epk_h.at[pl.ds(e0, 2 * EB)], in_v.at[j],
                             insem.at[j])
            f0 = (s * NB + b) * EB
            pltpu.async_copy(inv_h.at[pl.ds(f0, EB)], inf_v.at[j],
                             insem.at[j])

        def wait_in(b, j):
            e0 = (s * NB + b) * (2 * EB)
            pltpu.make_async_copy(epk_h.at[pl.ds(e0, 2 * EB)], in_v.at[j],
                                  insem.at[j]).wait()
            f0 = (s * NB + b) * EB
            pltpu.make_async_copy(inv_h.at[pl.ds(f0, EB)], inf_v.at[j],
                                  insem.at[j]).wait()

        def prep(j):
            for kk in range(EB // L):
                sl = pl.ds(kk * L, L)
                idx_v[j, sl] = in_v[j, sl] + c
                sidx_v[j, sl] = in_v[j, pl.ds(EB + kk * L, L)]

        def issue_gather(j):
            pltpu.async_copy(tbl_h.at[idx_v.at[j]], rows_v.at[j % 2],
                             gsem.at[j])

        def wait_gather(j):
            pltpu.make_async_copy(tbl_h.at[idx_v.at[j]], rows_v.at[j % 2],
                                  gsem.at[j]).wait()

        def scale(j):
            j2 = j % 2
            for kk in range(EB // L):
                iv = inf_v[j, pl.ds(kk * L, L)]
                for jj in range(L):
                    e = kk * L + jj
                    bc = _lane_bcast(iv, jj)
                    srow_v[j2, e, pl.ds(0, L)] = \
                        rows_v[j2, e, pl.ds(0, L)] * bc
                    srow_v[j2, e, pl.ds(L, L)] = \
                        rows_v[j2, e, pl.ds(L, L)] * bc

        def issue_scatter(j):
            pltpu.async_copy(srow_v.at[j % 2], agg_sh.at[sidx_v.at[j]],
                             ssem.at[j], add=True)

        def wait_scatter(j):
            pltpu.make_async_copy(srow_v.at[j % 2], agg_sh.at[sidx_v.at[j]],
                                  ssem.at[j]).wait()

        issue_in(0, 0)
        issue_in(1, 1)
        issue_in(2, 2)
        wait_in(0, 0)
        prep(0)
        issue_gather(0)

        def body(g, _):
            for j in range(4):
                b = 4 * g + j
                jn = (j + 1) % 4
                wait_in(b + 1, jn)
                prep(jn)
                issue_gather(jn)
                wait_gather(j)

                @pl.when(b >= 2)
                def _():
                    wait_scatter((j + 2) % 4)

                scale(j)
                issue_scatter(j)

                @pl.when(b < NB - 3)
                def _():
                    issue_in(b + 3, (j + 3) % 4)
            return ()

        lax.fori_loop(0, NB // 4 - 1, body, ())
        issue_in(NB - 1, (NB - 1) % 4)
        for j in range(4):
            if j < 3:
                wait_in(NB - 3 + j, (j + 1) % 4)
                prep((j + 1) % 4)
                issue_gather((j + 1) % 4)
            wait_gather(j)
            wait_scatter((j + 2) % 4)
            scale(j)
            issue_scatter(j)
        wait_scatter(2)
        wait_scatter(3)

        plsc.subcore_barrier()
        pltpu.sync_copy(
            agg_sh.at[pl.ds(s * PER_S, PER_S)],
            out_h.at[c, pl.ds(s * PER_S, PER_S)],
        )

    return k(tbl, epk, invE, zblk)


def _ln_gelu_body(h_ref, g_ref, b_ref, o_ref):
    h = h_ref[...]
    m = jnp.mean(h, axis=-1, keepdims=True)
    v = jnp.mean((h - m) * (h - m), axis=-1, keepdims=True)
    xn = (h - m) * jax.lax.rsqrt(v + 1e-5) * g_ref[...] + b_ref[...]
    o_ref[...] = xn * 0.5 * (1.0 + jax.lax.erf(xn / jnp.sqrt(2.0).astype(xn.dtype)))


def _ln_gelu(h, g, b):
    grid = (N_NODES // _ROW_TILE,)
    return pl.pallas_call(
        _ln_gelu_body,
        grid=grid,
        in_specs=[
            pl.BlockSpec((_ROW_TILE, D), lambda i: (i, 0)),
            pl.BlockSpec((1, D), lambda i: (0, 0)),
            pl.BlockSpec((1, D), lambda i: (0, 0)),
        ],
        out_specs=pl.BlockSpec((_ROW_TILE, D), lambda i: (i, 0)),
        out_shape=jax.ShapeDtypeStruct((N_NODES, D), jnp.float32),
    )(h, g.reshape(1, D), b.reshape(1, D))


def _rgcn_conv(y, epk, invE, zagg, W, root, b):
    xW = jnp.einsum('ni,rio->rno', y, W)
    tbl = xW.reshape(R * N_NODES * NC, DH)
    agg3 = _sc_conv(tbl, epk, invE, zagg)
    agg = jnp.concatenate([agg3[0][:N_NODES], agg3[1][:N_NODES]], axis=-1)
    return agg + y @ root + b


def kernel(x, edge_index, edge_attr, params):
    src = edge_index[0].astype(jnp.int32)
    dst = edge_index[1].astype(jnp.int32)
    et = edge_attr.astype(jnp.int32)

    pad = EPAD - N_EDGES
    comb = jnp.concatenate([dst * R + et, jnp.full((pad,), N_NODES * R, jnp.int32)])
    g2 = jnp.concatenate([(et * N_NODES + src) * NC, jnp.zeros((pad,), jnp.int32)])
    dstp = jnp.concatenate([dst, jnp.full((pad,), N_NODES, jnp.int32)])
    val = jnp.concatenate([jnp.ones((N_EDGES,), jnp.float32),
                           jnp.zeros((pad,), jnp.float32)])

    zcnt = jnp.zeros((CNT_PER_S,), jnp.float32)
    zagg = jnp.zeros((PER_S, DH), jnp.float32)

    cnt = _sc_count(comb, val, zcnt)
    invp = 1.0 / jnp.maximum(cnt, 1.0)
    invE = _sc_inv_gather(comb, invp)
    epk = jnp.stack([g2.reshape(-1, EB), dstp.reshape(-1, EB)],
                    axis=1).reshape(-1)

    emb = params['emb']
    h = jnp.where((x[:, None] == 1), emb[1][None, :], emb[0][None, :])

    for p in params['blocks']:
        y = _ln_gelu(h, p['ln1_g'], p['ln1_b'])
        out = _rgcn_conv(y, epk, invE, zagg, p['W1'], p['root1'], p['b1'])
        y2 = _ln_gelu(out, p['ln2_g'], p['ln2_b'])
        out2 = _rgcn_conv(y2, epk, invE, zagg, p['W2'], p['root2'], p['b2'])
        h = out2 + h

    h = h @ params['cr_W'] + params['cr_b']
    pooled = jnp.mean(h, axis=0)
    z = jax.nn.gelu(pooled @ params['p1_W'] + params['p1_b'], approximate=False)
    return z @ params['p2_W'] + params['p2_b']
